# baseline (speedup 1.0000x reference)
import functools

import jax
import jax.numpy as jnp
from jax import lax
from jax.experimental import pallas as pl
from jax.experimental.pallas import tpu as pltpu
from jax.experimental.pallas import tpu_sc as plsc

TN = 512
TM = 512
CH = 32


def _three_nn_body(x1_ref, x2t_ref, gidx_ref, w_ref, *, M):
    b = pl.program_id(0)
    x1 = x1_ref[0]
    x2 = x2t_ref[0]
    n1 = jnp.sum(x1 * x1, axis=1, keepdims=True)
    n2 = jnp.sum(x2 * x2, axis=0, keepdims=True)
    dot = lax.dot_general(x1, x2, (((1,), (0,)), ((), ())),
                          precision=lax.Precision.HIGHEST,
                          preferred_element_type=jnp.float32)
    d2 = jnp.maximum(n1 + n2 - 2.0 * dot, 0.0)
    iota = lax.broadcasted_iota(jnp.int32, d2.shape, 1)
    keys = jnp.bitwise_or(
        jnp.bitwise_and(lax.bitcast_convert_type(d2, jnp.int32),
                        jnp.int32(-1024)), iota)
    idx_cols = []
    rec_cols = []
    work = keys
    for j in range(3):
        kmin = jnp.min(work, axis=1, keepdims=True)
        if j < 2:
            work = jnp.where(work == kmin, jnp.int32(0x7FFFFFFF), work)
        idx_cols.append(jnp.bitwise_and(kmin, jnp.int32(M - 1)))
        dval = lax.bitcast_convert_type(
            jnp.bitwise_and(kmin, jnp.int32(-1024)), jnp.float32)
        rec_cols.append(1.0 / jnp.maximum(dval, 1e-10))
    norm = rec_cols[0] + rec_cols[1] + rec_cols[2]
    w_ref[0] = jnp.concatenate([r / norm for r in rec_cols], axis=1)
    gidx_ref[0] = jnp.concatenate(idx_cols, axis=1) + b * M


def _three_nn(xyz1, xyz2t):
    B, N, _ = xyz1.shape
    M = xyz2t.shape[2]
    grid = (B, N // TN)
    return pl.pallas_call(
        functools.partial(_three_nn_body, M=M),
        grid=grid,
        in_specs=[
            pl.BlockSpec((1, TN, 3), lambda b, n: (b, n, 0)),
            pl.BlockSpec((1, 3, M), lambda b, n: (b, 0, 0)),
        ],
        out_specs=[
            pl.BlockSpec((1, TN, 3), lambda b, n: (b, n, 0)),
            pl.BlockSpec((1, TN, 3), lambda b, n: (b, n, 0)),
        ],
        out_shape=[
            jax.ShapeDtypeStruct((B, N, 3), jnp.int32),
            jax.ShapeDtypeStruct((B, N, 3), jnp.float32),
        ],
        compiler_params=pltpu.CompilerParams(
            dimension_semantics=("parallel", "parallel")),
    )(xyz1, xyz2t)


def _sc_interpolate(gidx2d, w_flat, table):
    BN3 = w_flat.shape[0]
    BN = BN3 // 3
    C2 = table.shape[1]
    info = plsc.get_sparse_core_info()
    nw = info.num_cores * info.num_subcores
    rows_per_w = BN // nw
    n_chunks = rows_per_w // CH
    wpad = 3 * rows_per_w + 16
    mesh = plsc.VectorSubcoreMesh(core_axis_name="c", subcore_axis_name="s")

    @functools.partial(
        pl.kernel,
        mesh=mesh,
        out_type=jax.ShapeDtypeStruct((BN, C2), jnp.float32),
        scratch_types=[
            pltpu.VMEM((n_chunks, 3 * CH), jnp.int32),
            pltpu.VMEM((wpad,), jnp.float32),
            pltpu.VMEM((4, 3 * CH, C2), jnp.float32),
            pltpu.VMEM((2, CH, C2), jnp.float32),
            pltpu.SemaphoreType.DMA,
            pltpu.SemaphoreType.DMA,
            pltpu.SemaphoreType.DMA,
            pltpu.SemaphoreType.DMA,
            pltpu.SemaphoreType.DMA,
            pltpu.SemaphoreType.DMA,
            pltpu.SemaphoreType.DMA,
        ],
    )
    def k(gidx_hbm, w_hbm, table_hbm, out_hbm, idx_v, wv, rows_v, out_v,
          psem, gsem0, gsem1, gsem2, gsem3, osem0, osem1):
        wid = lax.axis_index("s") * info.num_cores + lax.axis_index("c")
        base = wid * rows_per_w

        pltpu.async_copy(gidx_hbm.at[pl.ds(wid * n_chunks, n_chunks)],
                         idx_v, psem)
        pltpu.async_copy(w_hbm.at[pl.ds(3 * base, 3 * rows_per_w)],
                         wv.at[pl.ds(0, 3 * rows_per_w)], psem).wait()
        pltpu.make_async_copy(gidx_hbm.at[pl.ds(0, n_chunks)], idx_v,
                              psem).wait()

        gsems = (gsem0, gsem1, gsem2, gsem3)
        osems = (osem0, osem1)

        def issue_gather(c, b):
            pltpu.async_copy(table_hbm.at[idx_v.at[c]], rows_v.at[b],
                             gsems[b])

        def wait_gather(c, b):
            pltpu.make_async_copy(table_hbm.at[idx_v.at[c]], rows_v.at[b],
                                  gsems[b]).wait()

        def issue_out(c, b):
            pltpu.async_copy(out_v.at[b],
                             out_hbm.at[pl.ds(base + c * CH, CH)], osems[b])

        def wait_out(c, b):
            pltpu.make_async_copy(out_v.at[b],
                                  out_hbm.at[pl.ds(base + c * CH, CH)],
                                  osems[b]).wait()

        issue_gather(0, 0)
        issue_gather(1, 1)
        issue_gather(2, 2)

        @pl.loop(0, n_chunks, step=4)
        def group(g):
            for b in range(4):
                c = g + b
                ob = b % 2

                @pl.when(c + 3 < n_chunks)
                def _():
                    issue_gather(c + 3, (b + 3) % 4)

                wait_gather(c, b)

                @pl.when(c >= 2)
                def _():
                    wait_out(c - 2, ob)

                def make_point(c=c, b=b, ob=ob):
                    @plsc.parallel_loop(0, CH, unroll=8)
                    def point(p):
                        woff = 3 * (c * CH + p)
                        wvec = wv[pl.ds(woff, 16)]
                        w0 = wvec[0]
                        w1 = wvec[1]
                        w2 = wvec[2]
                        for gg in range(C2 // 16):
                            s = pl.ds(gg * 16, 16)
                            out_v[ob, p, s] = (w0 * rows_v[b, 3 * p, s]
                                               + w1 * rows_v[b, 3 * p + 1, s]
                                               + w2 * rows_v[b, 3 * p + 2, s])

                make_point()
                issue_out(c, ob)

        wait_out(n_chunks - 2, 0)
        wait_out(n_chunks - 1, 1)

    return k(gidx2d, w_flat, table)


def _mlp_body(f1_ref, it_ref, w1a_ref, w1b_ref, b1_ref, w2_ref, b2_ref,
              w3_ref, b3_ref, out_ref):
    h = jnp.dot(f1_ref[...], w1a_ref[...], preferred_element_type=jnp.float32)
    h = h + jnp.dot(it_ref[...], w1b_ref[...], preferred_element_type=jnp.float32)
    h = jnp.maximum(h + b1_ref[...], 0.0)
    h = jnp.maximum(jnp.dot(h, w2_ref[...], preferred_element_type=jnp.float32)
                    + b2_ref[...], 0.0)
    out_ref[...] = (jnp.dot(h, w3_ref[...], preferred_element_type=jnp.float32)
                    + b3_ref[...])


def _mlp(f1r, interp, W1a, W1b, b1, W2, b2, W3, b3):
    BN, C1 = f1r.shape
    C2 = interp.shape[1]
    h0 = W1a.shape[1]
    h1 = W2.shape[1]
    h2 = W3.shape[1]
    grid = (BN // TM,)
    full = lambda shape: pl.BlockSpec(shape, lambda i: tuple(0 for _ in shape))
    return pl.pallas_call(
        _mlp_body,
        grid=grid,
        in_specs=[
            pl.BlockSpec((TM, C1), lambda i: (i, 0)),
            pl.BlockSpec((TM, C2), lambda i: (i, 0)),
            full((C1, h0)),
            full((C2, h0)),
            full((1, h0)),
            full((h0, h1)),
            full((1, h1)),
            full((h1, h2)),
            full((1, h2)),
        ],
        out_specs=pl.BlockSpec((TM, h2), lambda i: (i, 0)),
        out_shape=jax.ShapeDtypeStruct((BN, h2), jnp.float32),
        compiler_params=pltpu.CompilerParams(
            dimension_semantics=("parallel",)),
    )(f1r, interp, W1a, W1b, b1, W2, b2, W3, b3)


def kernel(xyz1, xyz2, features1, features2, W1, b1, W2, b2, W3, b3):
    B, N, _ = xyz1.shape
    M = xyz2.shape[1]
    C1 = features1.shape[2]
    C2 = features2.shape[2]

    gidx, w = _three_nn(xyz1, jnp.transpose(xyz2, (0, 2, 1)))

    interp = _sc_interpolate(
        gidx.reshape(B * N * 3 // (3 * CH), 3 * CH),
        w.reshape(B * N * 3),
        features2.reshape(B * M, C2),
    )

    out = _mlp(
        features1.reshape(B * N, C1),
        interp,
        W1[:C1], W1[C1:],
        b1.reshape(1, -1),
        W2, b2.reshape(1, -1),
        W3, b3.reshape(1, -1),
    )
    return out.reshape(B, N, W3.shape[1])


# speedup vs baseline: 1.1907x; 1.1907x over previous
import functools

import jax
import jax.numpy as jnp
from jax import lax
from jax.experimental import pallas as pl
from jax.experimental.pallas import tpu as pltpu
from jax.experimental.pallas import tpu_sc as plsc

TN = 512
TM = 512
CH = 32


def _three_nn_body(x1_ref, x2t_ref, gidx_ref, w_ref, *, M):
    b = pl.program_id(0)
    x1 = x1_ref[0]
    x2 = x2t_ref[0]
    d2 = jnp.zeros((x1.shape[0], M), jnp.float32)
    for c in range(3):
        diff = x1[:, c:c + 1] - x2[c:c + 1, :]
        d2 = d2 + diff * diff
    iota = lax.broadcasted_iota(jnp.int32, d2.shape, 1)
    keys = jnp.bitwise_or(
        jnp.bitwise_and(lax.bitcast_convert_type(d2, jnp.int32),
                        jnp.int32(-1024)), iota)
    idx_cols = []
    rec_cols = []
    work = keys
    for j in range(3):
        kmin = jnp.min(work, axis=1, keepdims=True)
        if j < 2:
            work = jnp.where(work == kmin, jnp.int32(0x7FFFFFFF), work)
        idx_cols.append(jnp.bitwise_and(kmin, jnp.int32(M - 1)))
        dval = lax.bitcast_convert_type(
            jnp.bitwise_and(kmin, jnp.int32(-1024)), jnp.float32)
        rec_cols.append(1.0 / jnp.maximum(dval, 1e-10))
    norm = rec_cols[0] + rec_cols[1] + rec_cols[2]
    w_ref[0] = jnp.concatenate([r / norm for r in rec_cols], axis=1)
    gidx_ref[0] = jnp.concatenate(idx_cols, axis=1) + b * M


def _three_nn(xyz1, xyz2t):
    B, N, _ = xyz1.shape
    M = xyz2t.shape[2]
    grid = (B, N // TN)
    return pl.pallas_call(
        functools.partial(_three_nn_body, M=M),
        grid=grid,
        in_specs=[
            pl.BlockSpec((1, TN, 3), lambda b, n: (b, n, 0)),
            pl.BlockSpec((1, 3, M), lambda b, n: (b, 0, 0)),
        ],
        out_specs=[
            pl.BlockSpec((1, TN, 3), lambda b, n: (b, n, 0)),
            pl.BlockSpec((1, TN, 3), lambda b, n: (b, n, 0)),
        ],
        out_shape=[
            jax.ShapeDtypeStruct((B, N, 3), jnp.int32),
            jax.ShapeDtypeStruct((B, N, 3), jnp.float32),
        ],
        compiler_params=pltpu.CompilerParams(
            dimension_semantics=("parallel", "parallel")),
    )(xyz1, xyz2t)


def _sc_interpolate(gidx2d, w_flat, table):
    BN3 = w_flat.shape[0]
    BN = BN3 // 3
    C2 = table.shape[1]
    info = plsc.get_sparse_core_info()
    nw = info.num_cores * info.num_subcores
    rows_per_w = BN // nw
    n_chunks = rows_per_w // CH
    wpad = 3 * rows_per_w + 16
    mesh = plsc.VectorSubcoreMesh(core_axis_name="c", subcore_axis_name="s")

    @functools.partial(
        pl.kernel,
        mesh=mesh,
        out_type=jax.ShapeDtypeStruct((BN, C2), jnp.float32),
        scratch_types=[
            pltpu.VMEM((n_chunks, 3 * CH), jnp.int32),
            pltpu.VMEM((wpad,), jnp.float32),
            pltpu.VMEM((4, 3 * CH, C2), jnp.float32),
            pltpu.VMEM((2, CH, C2), jnp.float32),
            pltpu.SemaphoreType.DMA,
            pltpu.SemaphoreType.DMA,
            pltpu.SemaphoreType.DMA,
            pltpu.SemaphoreType.DMA,
            pltpu.SemaphoreType.DMA,
            pltpu.SemaphoreType.DMA,
            pltpu.SemaphoreType.DMA,
        ],
    )
    def k(gidx_hbm, w_hbm, table_hbm, out_hbm, idx_v, wv, rows_v, out_v,
          psem, gsem0, gsem1, gsem2, gsem3, osem0, osem1):
        wid = lax.axis_index("s") * info.num_cores + lax.axis_index("c")
        base = wid * rows_per_w

        pltpu.async_copy(gidx_hbm.at[pl.ds(wid * n_chunks, n_chunks)],
                         idx_v, psem)
        pltpu.async_copy(w_hbm.at[pl.ds(3 * base, 3 * rows_per_w)],
                         wv.at[pl.ds(0, 3 * rows_per_w)], psem).wait()
        pltpu.make_async_copy(gidx_hbm.at[pl.ds(0, n_chunks)], idx_v,
                              psem).wait()

        gsems = (gsem0, gsem1, gsem2, gsem3)
        osems = (osem0, osem1)

        def issue_gather(c, b):
            pltpu.async_copy(table_hbm.at[idx_v.at[c]], rows_v.at[b],
                             gsems[b])

        def wait_gather(c, b):
            pltpu.make_async_copy(table_hbm.at[idx_v.at[c]], rows_v.at[b],
                                  gsems[b]).wait()

        def issue_out(c, b):
            pltpu.async_copy(out_v.at[b],
                             out_hbm.at[pl.ds(base + c * CH, CH)], osems[b])

        def wait_out(c, b):
            pltpu.make_async_copy(out_v.at[b],
                                  out_hbm.at[pl.ds(base + c * CH, CH)],
                                  osems[b]).wait()

        issue_gather(0, 0)
        issue_gather(1, 1)
        issue_gather(2, 2)

        @pl.loop(0, n_chunks, step=4)
        def group(g):
            for b in range(4):
                c = g + b
                ob = b % 2

                @pl.when(c + 3 < n_chunks)
                def _():
                    issue_gather(c + 3, (b + 3) % 4)

                wait_gather(c, b)

                @pl.when(c >= 2)
                def _():
                    wait_out(c - 2, ob)

                def make_point(c=c, b=b, ob=ob):
                    @plsc.parallel_loop(0, CH, unroll=8)
                    def point(p):
                        woff = 3 * (c * CH + p)
                        wvec = wv[pl.ds(woff, 16)]
                        w0 = wvec[0]
                        w1 = wvec[1]
                        w2 = wvec[2]
                        for gg in range(C2 // 16):
                            s = pl.ds(gg * 16, 16)
                            out_v[ob, p, s] = (w0 * rows_v[b, 3 * p, s]
                                               + w1 * rows_v[b, 3 * p + 1, s]
                                               + w2 * rows_v[b, 3 * p + 2, s])

                make_point()
                issue_out(c, ob)

        wait_out(n_chunks - 2, 0)
        wait_out(n_chunks - 1, 1)

    return k(gidx2d, w_flat, table)


def _mlp_body(f1_ref, it_ref, w1a_ref, w1b_ref, b1_ref, w2_ref, b2_ref,
              w3_ref, b3_ref, out_ref):
    h = jnp.dot(f1_ref[...], w1a_ref[...], preferred_element_type=jnp.float32)
    h = h + jnp.dot(it_ref[...], w1b_ref[...], preferred_element_type=jnp.float32)
    h = jnp.maximum(h + b1_ref[...], 0.0)
    h = jnp.maximum(jnp.dot(h, w2_ref[...], preferred_element_type=jnp.float32)
                    + b2_ref[...], 0.0)
    out_ref[...] = (jnp.dot(h, w3_ref[...], preferred_element_type=jnp.float32)
                    + b3_ref[...])


def _mlp(f1r, interp, W1a, W1b, b1, W2, b2, W3, b3):
    BN, C1 = f1r.shape
    C2 = interp.shape[1]
    h0 = W1a.shape[1]
    h1 = W2.shape[1]
    h2 = W3.shape[1]
    grid = (BN // TM,)
    full = lambda shape: pl.BlockSpec(shape, lambda i: tuple(0 for _ in shape))
    return pl.pallas_call(
        _mlp_body,
        grid=grid,
        in_specs=[
            pl.BlockSpec((TM, C1), lambda i: (i, 0)),
            pl.BlockSpec((TM, C2), lambda i: (i, 0)),
            full((C1, h0)),
            full((C2, h0)),
            full((1, h0)),
            full((h0, h1)),
            full((1, h1)),
            full((h1, h2)),
            full((1, h2)),
        ],
        out_specs=pl.BlockSpec((TM, h2), lambda i: (i, 0)),
        out_shape=jax.ShapeDtypeStruct((BN, h2), jnp.float32),
        compiler_params=pltpu.CompilerParams(
            dimension_semantics=("parallel",)),
    )(f1r, interp, W1a, W1b, b1, W2, b2, W3, b3)


def kernel(xyz1, xyz2, features1, features2, W1, b1, W2, b2, W3, b3):
    B, N, _ = xyz1.shape
    M = xyz2.shape[1]
    C1 = features1.shape[2]
    C2 = features2.shape[2]

    gidx, w = _three_nn(xyz1, jnp.transpose(xyz2, (0, 2, 1)))

    interp = _sc_interpolate(
        gidx.reshape(B * N * 3 // (3 * CH), 3 * CH),
        w.reshape(B * N * 3),
        features2.reshape(B * M, C2),
    )

    out = _mlp(
        features1.reshape(B * N, C1),
        interp,
        W1[:C1], W1[C1:],
        b1.reshape(1, -1),
        W2, b2.reshape(1, -1),
        W3, b3.reshape(1, -1),
    )
    return out.reshape(B, N, W3.shape[1])


# speedup vs baseline: 1.2415x; 1.0426x over previous
import functools

import jax
import jax.numpy as jnp
from jax import lax
from jax.experimental import pallas as pl
from jax.experimental.pallas import tpu as pltpu
from jax.experimental.pallas import tpu_sc as plsc

TN = 512
TM = 512
CH = 32


def _three_nn_body(x1_ref, x2t_ref, gidx_ref, w_ref, *, M):
    b = pl.program_id(0)
    x1 = x1_ref[0]
    x2 = x2t_ref[0]
    d2 = jnp.zeros((x1.shape[0], M), jnp.float32)
    for c in range(3):
        diff = x1[:, c:c + 1] - x2[c:c + 1, :]
        d2 = d2 + diff * diff
    iota = lax.broadcasted_iota(jnp.int32, d2.shape, 1)
    keys = jnp.bitwise_or(
        jnp.bitwise_and(lax.bitcast_convert_type(d2, jnp.int32),
                        jnp.int32(-1024)), iota)
    idx_cols = []
    rec_cols = []
    work = keys
    for j in range(3):
        kmin = jnp.min(work, axis=1, keepdims=True)
        if j < 2:
            work = jnp.where(work == kmin, jnp.int32(0x7FFFFFFF), work)
        idx_cols.append(jnp.bitwise_and(kmin, jnp.int32(M - 1)))
        dval = lax.bitcast_convert_type(
            jnp.bitwise_and(kmin, jnp.int32(-1024)), jnp.float32)
        rec_cols.append(1.0 / jnp.maximum(dval, 1e-10))
    norm = rec_cols[0] + rec_cols[1] + rec_cols[2]
    w_ref[0] = jnp.concatenate([r / norm for r in rec_cols], axis=1)
    gidx_ref[0] = jnp.concatenate(idx_cols, axis=1) + b * M


def _three_nn(xyz1, xyz2t):
    B, N, _ = xyz1.shape
    M = xyz2t.shape[2]
    grid = (B, N // TN)
    return pl.pallas_call(
        functools.partial(_three_nn_body, M=M),
        grid=grid,
        in_specs=[
            pl.BlockSpec((1, TN, 3), lambda b, n: (b, n, 0)),
            pl.BlockSpec((1, 3, M), lambda b, n: (b, 0, 0)),
        ],
        out_specs=[
            pl.BlockSpec((1, TN, 3), lambda b, n: (b, n, 0)),
            pl.BlockSpec((1, TN, 3), lambda b, n: (b, n, 0)),
        ],
        out_shape=[
            jax.ShapeDtypeStruct((B, N, 3), jnp.int32),
            jax.ShapeDtypeStruct((B, N, 3), jnp.float32),
        ],
        compiler_params=pltpu.CompilerParams(
            dimension_semantics=("parallel", "parallel")),
    )(xyz1, xyz2t)


def _sc_interpolate(gidx2d, w_flat, table):
    BN3 = w_flat.shape[0]
    BN = BN3 // 3
    C2 = table.shape[1]
    info = plsc.get_sparse_core_info()
    nw = info.num_cores * info.num_subcores
    rows_per_w = BN // nw
    n_chunks = rows_per_w // CH
    wpad = 3 * rows_per_w + 16
    mesh = plsc.VectorSubcoreMesh(core_axis_name="c", subcore_axis_name="s")

    @functools.partial(
        pl.kernel,
        mesh=mesh,
        out_type=jax.ShapeDtypeStruct((BN, C2), jnp.float32),
        scratch_types=[
            pltpu.VMEM((n_chunks, 3 * CH), jnp.int32),
            pltpu.VMEM((wpad,), jnp.float32),
            pltpu.VMEM((4, 3 * CH, C2), jnp.float32),
            pltpu.VMEM((2, CH, C2), jnp.float32),
            pltpu.SemaphoreType.DMA,
            pltpu.SemaphoreType.DMA,
            pltpu.SemaphoreType.DMA,
            pltpu.SemaphoreType.DMA,
            pltpu.SemaphoreType.DMA,
            pltpu.SemaphoreType.DMA,
            pltpu.SemaphoreType.DMA,
        ],
    )
    def k(gidx_hbm, w_hbm, table_hbm, out_hbm, idx_v, wv, rows_v, out_v,
          psem, gsem0, gsem1, gsem2, gsem3, osem0, osem1):
        wid = lax.axis_index("s") * info.num_cores + lax.axis_index("c")
        base = wid * rows_per_w

        pltpu.async_copy(gidx_hbm.at[pl.ds(wid * n_chunks, n_chunks)],
                         idx_v, psem)
        pltpu.async_copy(w_hbm.at[pl.ds(3 * base, 3 * rows_per_w)],
                         wv.at[pl.ds(0, 3 * rows_per_w)], psem).wait()
        pltpu.make_async_copy(gidx_hbm.at[pl.ds(0, n_chunks)], idx_v,
                              psem).wait()

        gsems = (gsem0, gsem1, gsem2, gsem3)
        osems = (osem0, osem1)

        def issue_gather(c, b):
            pltpu.async_copy(table_hbm.at[idx_v.at[c]], rows_v.at[b],
                             gsems[b])

        def wait_gather(c, b):
            pltpu.make_async_copy(table_hbm.at[idx_v.at[c]], rows_v.at[b],
                                  gsems[b]).wait()

        def issue_out(c, b):
            pltpu.async_copy(out_v.at[b],
                             out_hbm.at[pl.ds(base + c * CH, CH)], osems[b])

        def wait_out(c, b):
            pltpu.make_async_copy(out_v.at[b],
                                  out_hbm.at[pl.ds(base + c * CH, CH)],
                                  osems[b]).wait()

        issue_gather(0, 0)
        issue_gather(1, 1)
        issue_gather(2, 2)

        @pl.loop(0, n_chunks, step=4)
        def group(g):
            for b in range(4):
                c = g + b
                ob = b % 2

                @pl.when(c + 3 < n_chunks)
                def _():
                    issue_gather(c + 3, (b + 3) % 4)

                wait_gather(c, b)

                @pl.when(c >= 2)
                def _():
                    wait_out(c - 2, ob)

                def make_point(c=c, b=b, ob=ob):
                    @plsc.parallel_loop(0, CH, unroll=8)
                    def point(p):
                        woff = 3 * (c * CH + p)
                        wvec = wv[pl.ds(woff, 16)]
                        w0 = wvec[0]
                        w1 = wvec[1]
                        w2 = wvec[2]
                        for gg in range(C2 // 16):
                            s = pl.ds(gg * 16, 16)
                            out_v[ob, p, s] = (w0 * rows_v[b, 3 * p, s]
                                               + w1 * rows_v[b, 3 * p + 1, s]
                                               + w2 * rows_v[b, 3 * p + 2, s])

                make_point()
                issue_out(c, ob)

        wait_out(n_chunks - 2, 0)
        wait_out(n_chunks - 1, 1)

    return k(gidx2d, w_flat, table)


def _mlp_body(f1_ref, it_ref, w1a_ref, w1b_ref, b1_ref, w2_ref, b2_ref,
              w3_ref, b3_ref, out_ref):
    h = jnp.dot(f1_ref[...], w1a_ref[...], preferred_element_type=jnp.float32)
    h = h + jnp.dot(it_ref[...], w1b_ref[...], preferred_element_type=jnp.float32)
    h = jnp.maximum(h + b1_ref[...], 0.0)
    h = jnp.maximum(jnp.dot(h, w2_ref[...], preferred_element_type=jnp.float32)
                    + b2_ref[...], 0.0)
    out_ref[...] = (jnp.dot(h, w3_ref[...], preferred_element_type=jnp.float32)
                    + b3_ref[...])


def _mlp(f1r, interp, W1a, W1b, b1, W2, b2, W3, b3):
    BN, C1 = f1r.shape
    C2 = interp.shape[1]
    h0 = W1a.shape[1]
    h1 = W2.shape[1]
    h2 = W3.shape[1]
    grid = (BN // TM,)
    full = lambda shape: pl.BlockSpec(shape, lambda i: tuple(0 for _ in shape))
    return pl.pallas_call(
        _mlp_body,
        grid=grid,
        in_specs=[
            pl.BlockSpec((TM, C1), lambda i: (i, 0)),
            pl.BlockSpec((TM, C2), lambda i: (i, 0)),
            full((C1, h0)),
            full((C2, h0)),
            full((1, h0)),
            full((h0, h1)),
            full((1, h1)),
            full((h1, h2)),
            full((1, h2)),
        ],
        out_specs=pl.BlockSpec((TM, h2), lambda i: (i, 0)),
        out_shape=jax.ShapeDtypeStruct((BN, h2), jnp.float32),
        compiler_params=pltpu.CompilerParams(
            dimension_semantics=("parallel",)),
    )(f1r, interp, W1a, W1b, b1, W2, b2, W3, b3)


def kernel(xyz1, xyz2, features1, features2, W1, b1, W2, b2, W3, b3):
    B, N, _ = xyz1.shape
    M = xyz2.shape[1]
    C1 = features1.shape[2]
    C2 = features2.shape[2]

    xyz2t = jnp.transpose(xyz2, (0, 2, 1))
    table = features2.reshape(B * M, C2)
    W1a, W1b = W1[:C1], W1[C1:]
    b1r, b2r, b3r = b1.reshape(1, -1), b2.reshape(1, -1), b3.reshape(1, -1)

    H = N // 2
    outs = []
    for h in range(2):
        x1h = xyz1[:, h * H:(h + 1) * H]
        gidx, w = _three_nn(x1h, xyz2t)
        interp = _sc_interpolate(
            gidx.reshape(B * H * 3 // (3 * CH), 3 * CH),
            w.reshape(B * H * 3),
            table,
        )
        f1h = features1[:, h * H:(h + 1) * H].reshape(B * H, C1)
        out = _mlp(f1h, interp, W1a, W1b, b1r, W2, b2r, W3, b3r)
        outs.append(out.reshape(B, H, W3.shape[1]))
    return jnp.concatenate(outs, axis=1)


# speedup vs baseline: 1.3507x; 1.0880x over previous
import functools

import jax
import jax.numpy as jnp
from jax import lax
from jax.experimental import pallas as pl
from jax.experimental.pallas import tpu as pltpu
from jax.experimental.pallas import tpu_sc as plsc

TN = 1024
TM = 1024
CH = 32


def _three_nn_body(x1_ref, x2t_ref, gidx_ref, w_ref, *, M):
    b = pl.program_id(0)
    x1 = x1_ref[0]
    x2 = x2t_ref[0]
    d2 = jnp.zeros((x1.shape[0], M), jnp.float32)
    for c in range(3):
        diff = x1[:, c:c + 1] - x2[c:c + 1, :]
        d2 = d2 + diff * diff
    iota = lax.broadcasted_iota(jnp.int32, d2.shape, 1)
    keys = jnp.bitwise_or(
        jnp.bitwise_and(lax.bitcast_convert_type(d2, jnp.int32),
                        jnp.int32(-1024)), iota)
    idx_cols = []
    rec_cols = []
    work = keys
    for j in range(3):
        kmin = jnp.min(work, axis=1, keepdims=True)
        if j < 2:
            work = jnp.where(work == kmin, jnp.int32(0x7FFFFFFF), work)
        idx_cols.append(jnp.bitwise_and(kmin, jnp.int32(M - 1)))
        dval = lax.bitcast_convert_type(
            jnp.bitwise_and(kmin, jnp.int32(-1024)), jnp.float32)
        rec_cols.append(1.0 / jnp.maximum(dval, 1e-10))
    norm = rec_cols[0] + rec_cols[1] + rec_cols[2]
    w_ref[0] = jnp.concatenate([r / norm for r in rec_cols], axis=1)
    gidx_ref[0] = jnp.concatenate(idx_cols, axis=1) + b * M


def _three_nn(xyz1, xyz2t):
    B, N, _ = xyz1.shape
    M = xyz2t.shape[2]
    grid = (B, N // TN)
    return pl.pallas_call(
        functools.partial(_three_nn_body, M=M),
        grid=grid,
        in_specs=[
            pl.BlockSpec((1, TN, 3), lambda b, n: (b, n, 0)),
            pl.BlockSpec((1, 3, M), lambda b, n: (b, 0, 0)),
        ],
        out_specs=[
            pl.BlockSpec((1, TN, 3), lambda b, n: (b, n, 0)),
            pl.BlockSpec((1, TN, 3), lambda b, n: (b, n, 0)),
        ],
        out_shape=[
            jax.ShapeDtypeStruct((B, N, 3), jnp.int32),
            jax.ShapeDtypeStruct((B, N, 3), jnp.float32),
        ],
        compiler_params=pltpu.CompilerParams(
            dimension_semantics=("parallel", "parallel")),
    )(xyz1, xyz2t)


def _sc_interpolate(gidx2d, w_flat, table):
    BN3 = w_flat.shape[0]
    BN = BN3 // 3
    C2 = table.shape[1]
    info = plsc.get_sparse_core_info()
    nw = info.num_cores * info.num_subcores
    rows_per_w = BN // nw
    n_chunks = rows_per_w // CH
    wpad = 3 * rows_per_w + 16
    mesh = plsc.VectorSubcoreMesh(core_axis_name="c", subcore_axis_name="s")

    @functools.partial(
        pl.kernel,
        mesh=mesh,
        out_type=jax.ShapeDtypeStruct((BN, C2), jnp.float32),
        scratch_types=[
            pltpu.VMEM((n_chunks, 3 * CH), jnp.int32),
            pltpu.VMEM((wpad,), jnp.float32),
            pltpu.VMEM((4, 3 * CH, C2), jnp.float32),
            pltpu.VMEM((2, CH, C2), jnp.float32),
            pltpu.SemaphoreType.DMA,
            pltpu.SemaphoreType.DMA,
            pltpu.SemaphoreType.DMA,
            pltpu.SemaphoreType.DMA,
            pltpu.SemaphoreType.DMA,
            pltpu.SemaphoreType.DMA,
            pltpu.SemaphoreType.DMA,
        ],
    )
    def k(gidx_hbm, w_hbm, table_hbm, out_hbm, idx_v, wv, rows_v, out_v,
          psem, gsem0, gsem1, gsem2, gsem3, osem0, osem1):
        wid = lax.axis_index("s") * info.num_cores + lax.axis_index("c")
        base = wid * rows_per_w

        pltpu.async_copy(gidx_hbm.at[pl.ds(wid * n_chunks, n_chunks)],
                         idx_v, psem)
        pltpu.async_copy(w_hbm.at[pl.ds(3 * base, 3 * rows_per_w)],
                         wv.at[pl.ds(0, 3 * rows_per_w)], psem).wait()
        pltpu.make_async_copy(gidx_hbm.at[pl.ds(0, n_chunks)], idx_v,
                              psem).wait()

        gsems = (gsem0, gsem1, gsem2, gsem3)
        osems = (osem0, osem1)

        def issue_gather(c, b):
            pltpu.async_copy(table_hbm.at[idx_v.at[c]], rows_v.at[b],
                             gsems[b])

        def wait_gather(c, b):
            pltpu.make_async_copy(table_hbm.at[idx_v.at[c]], rows_v.at[b],
                                  gsems[b]).wait()

        def issue_out(c, b):
            pltpu.async_copy(out_v.at[b],
                             out_hbm.at[pl.ds(base + c * CH, CH)], osems[b])

        def wait_out(c, b):
            pltpu.make_async_copy(out_v.at[b],
                                  out_hbm.at[pl.ds(base + c * CH, CH)],
                                  osems[b]).wait()

        issue_gather(0, 0)
        issue_gather(1, 1)
        issue_gather(2, 2)

        @pl.loop(0, n_chunks, step=4)
        def group(g):
            for b in range(4):
                c = g + b
                ob = b % 2

                @pl.when(c + 3 < n_chunks)
                def _():
                    issue_gather(c + 3, (b + 3) % 4)

                wait_gather(c, b)

                @pl.when(c >= 2)
                def _():
                    wait_out(c - 2, ob)

                def make_point(c=c, b=b, ob=ob):
                    @plsc.parallel_loop(0, CH, unroll=8)
                    def point(p):
                        woff = 3 * (c * CH + p)
                        wvec = wv[pl.ds(woff, 16)]
                        w0 = wvec[0]
                        w1 = wvec[1]
                        w2 = wvec[2]
                        for gg in range(C2 // 16):
                            s = pl.ds(gg * 16, 16)
                            out_v[ob, p, s] = (w0 * rows_v[b, 3 * p, s]
                                               + w1 * rows_v[b, 3 * p + 1, s]
                                               + w2 * rows_v[b, 3 * p + 2, s])

                make_point()
                issue_out(c, ob)

        wait_out(n_chunks - 2, 0)
        wait_out(n_chunks - 1, 1)

    return k(gidx2d, w_flat, table)


def _mlp_body(f1_ref, it_ref, w1a_ref, w1b_ref, b1_ref, w2_ref, b2_ref,
              w3_ref, b3_ref, out_ref):
    h = jnp.dot(f1_ref[...], w1a_ref[...], preferred_element_type=jnp.float32)
    h = h + jnp.dot(it_ref[...], w1b_ref[...], preferred_element_type=jnp.float32)
    h = jnp.maximum(h + b1_ref[...], 0.0)
    h = jnp.maximum(jnp.dot(h, w2_ref[...], preferred_element_type=jnp.float32)
                    + b2_ref[...], 0.0)
    out_ref[...] = (jnp.dot(h, w3_ref[...], preferred_element_type=jnp.float32)
                    + b3_ref[...])


def _mlp(f1r, interp, W1a, W1b, b1, W2, b2, W3, b3):
    BN, C1 = f1r.shape
    C2 = interp.shape[1]
    h0 = W1a.shape[1]
    h1 = W2.shape[1]
    h2 = W3.shape[1]
    grid = (BN // TM,)
    full = lambda shape: pl.BlockSpec(shape, lambda i: tuple(0 for _ in shape))
    return pl.pallas_call(
        _mlp_body,
        grid=grid,
        in_specs=[
            pl.BlockSpec((TM, C1), lambda i: (i, 0)),
            pl.BlockSpec((TM, C2), lambda i: (i, 0)),
            full((C1, h0)),
            full((C2, h0)),
            full((1, h0)),
            full((h0, h1)),
            full((1, h1)),
            full((h1, h2)),
            full((1, h2)),
        ],
        out_specs=pl.BlockSpec((TM, h2), lambda i: (i, 0)),
        out_shape=jax.ShapeDtypeStruct((BN, h2), jnp.float32),
        compiler_params=pltpu.CompilerParams(
            dimension_semantics=("parallel",)),
    )(f1r, interp, W1a, W1b, b1, W2, b2, W3, b3)


def kernel(xyz1, xyz2, features1, features2, W1, b1, W2, b2, W3, b3):
    B, N, _ = xyz1.shape
    M = xyz2.shape[1]
    C1 = features1.shape[2]
    C2 = features2.shape[2]

    xyz2t = jnp.transpose(xyz2, (0, 2, 1))
    table = features2.reshape(B * M, C2)
    W1a, W1b = W1[:C1], W1[C1:]
    b1r, b2r, b3r = b1.reshape(1, -1), b2.reshape(1, -1), b3.reshape(1, -1)

    H = N // 2
    outs = []
    for h in range(2):
        x1h = xyz1[:, h * H:(h + 1) * H]
        gidx, w = _three_nn(x1h, xyz2t)
        interp = _sc_interpolate(
            gidx.reshape(B * H * 3 // (3 * CH), 3 * CH),
            w.reshape(B * H * 3),
            table,
        )
        f1h = features1[:, h * H:(h + 1) * H].reshape(B * H, C1)
        out = _mlp(f1h, interp, W1a, W1b, b1r, W2, b2r, W3, b3r)
        outs.append(out.reshape(B, H, W3.shape[1]))
    return jnp.concatenate(outs, axis=1)
